# baseline (speedup 1.0000x reference)
import jax
import jax.numpy as jnp
from jax import lax
from jax.experimental import pallas as pl
from jax.experimental.pallas import tpu as pltpu
from jax.experimental.pallas import tpu_sc as plsc

NC, NS = 2, 16
NW = NC * NS
HP = 56
CB = 2
CHUNK = CB * HP
EMB = 128


def _make_gather(batch, hist):
    mesh = plsc.VectorSubcoreMesh(core_axis_name="c", subcore_axis_name="s")
    b_per_w = batch // NW
    steps = b_per_w // CB
    idx_per_w = b_per_w * HP

    nbuf = 4
    lead = 2

    def body(idx_hbm, table_hbm, out_hbm, idx_v, rows_v, gsem, wsem):
        wid = lax.axis_index("s") * NC + lax.axis_index("c")
        pltpu.sync_copy(idx_hbm.at[wid], idx_v)
        b_base = wid * b_per_w

        def fire_gather(g, s):
            pltpu.async_copy(
                table_hbm.at[idx_v.at[pl.ds(g * CHUNK, CHUNK)]],
                rows_v.at[s], gsem,
            )

        def wait_gather():
            pltpu.make_async_copy(
                table_hbm.at[pl.ds(0, CHUNK)], rows_v.at[0], gsem
            ).wait()

        def fire_write(g, s):
            b0 = b_base + g * CB
            pltpu.async_copy(
                rows_v.at[s, pl.ds(0, hist)], out_hbm.at[b0], wsem)
            pltpu.async_copy(
                rows_v.at[s, pl.ds(HP, hist)], out_hbm.at[b0 + 1], wsem)

        def wait_write():
            pltpu.make_async_copy(
                rows_v.at[0, pl.ds(0, hist)], out_hbm.at[0], wsem
            ).wait()

        for g in range(lead):
            fire_gather(g, g % nbuf)

        def step(g, carry):
            s = g % nbuf
            wait_gather()
            fire_write(g, s)

            @pl.when(g >= nbuf - lead)
            def _():
                wait_write()
                wait_write()

            @pl.when(g + lead < steps)
            def _():
                fire_gather(g + lead, (g + lead) % nbuf)

            return carry

        lax.fori_loop(0, steps, step, 0)
        for _ in range(nbuf - lead):
            wait_write()
            wait_write()

    return pl.kernel(
        body,
        out_type=jax.ShapeDtypeStruct((batch, hist, EMB), jnp.float32),
        mesh=mesh,
        scratch_types=[
            pltpu.VMEM((idx_per_w,), jnp.int32),
            pltpu.VMEM((nbuf, CHUNK, EMB), jnp.float32),
            pltpu.SemaphoreType.DMA,
            pltpu.SemaphoreType.DMA,
        ],
        compiler_params=pltpu.CompilerParams(use_tc_tiling_on_sc=True),
    )


def kernel(x, table):
    B, H = x.shape
    xp = jnp.pad(x.astype(jnp.int32), ((0, 0), (0, HP - H)))
    idx = xp.reshape(NW, (B // NW) * HP)
    return _make_gather(B, H)(idx, table)


# speedup vs baseline: 15.6733x; 15.6733x over previous
import jax
import jax.numpy as jnp
from jax import lax
from jax.experimental import pallas as pl
from jax.experimental.pallas import tpu as pltpu
from jax.experimental.pallas import tpu_sc as plsc

NC, NS = 2, 16
NW = NC * NS
CHUNK = 128
EMB = 128


def _make_gather(batch, hist):
    mesh = plsc.VectorSubcoreMesh(core_axis_name="c", subcore_axis_name="s")
    total = batch * hist
    b_per_w = total // NW
    steps = b_per_w // CHUNK

    nbuf = 3
    WCH = 2 * CHUNK

    def body(idx_hbm, table_hbm, out_hbm, idx_v, rows_v, gsem, wsem):
        wid = lax.axis_index("s") * NC + lax.axis_index("c")
        pltpu.sync_copy(idx_hbm.at[wid], idx_v)
        out_flat = out_hbm.reshape(total, EMB)
        base = wid * b_per_w
        jsteps = steps // 2

        def fire_gather(c, slot_j):
            pltpu.async_copy(
                table_hbm.at[idx_v.at[c]],
                rows_v.at[slot_j % nbuf, pl.ds((c % 2) * CHUNK, CHUNK)], gsem)

        def fire_gather2(j):
            fire_gather(2 * j, j)
            fire_gather(2 * j + 1, j)

        def wait_gather2():
            for _ in range(2):
                pltpu.make_async_copy(
                    table_hbm.at[pl.ds(0, CHUNK)],
                    rows_v.at[0, pl.ds(0, CHUNK)], gsem).wait()

        def fire_write(j):
            pltpu.async_copy(
                rows_v.at[j % nbuf],
                out_flat.at[pl.ds(base + j * WCH, WCH)], wsem)

        def wait_write():
            pltpu.make_async_copy(
                rows_v.at[0], out_flat.at[pl.ds(base, WCH)], wsem).wait()

        fire_gather2(0)
        fire_gather2(1)
        wait_gather2()
        fire_write(0)
        fire_gather2(2)

        def step(j, carry):
            wait_gather2()
            fire_write(j)
            wait_write()
            fire_gather2(j + 2)
            return carry

        lax.fori_loop(1, jsteps - 2, step, 0)

        for j in range(jsteps - 2, jsteps):
            wait_gather2()
            fire_write(j)
            wait_write()
        wait_write()

    return pl.kernel(
        body,
        out_type=jax.ShapeDtypeStruct((hist, batch, EMB), jnp.float32),
        mesh=mesh,
        scratch_types=[
            pltpu.VMEM((steps, CHUNK), jnp.int32),
            pltpu.VMEM((nbuf, 2 * CHUNK, EMB), jnp.float32),
            pltpu.SemaphoreType.DMA,
            pltpu.SemaphoreType.DMA,
        ],
    )


def kernel(x, table):
    B, H = x.shape
    idx = x.T.astype(jnp.int32).reshape(NW, (B * H) // (NW * CHUNK), CHUNK)
    out_t = _make_gather(B, H)(idx, table)
    return out_t.transpose(1, 0, 2)
